# baseline (speedup 1.0000x reference)
import functools

import jax
import jax.numpy as jnp
from jax import lax
from jax.experimental import pallas as pl
from jax.experimental.pallas import tpu as pltpu
from jax.experimental.pallas import tpu_sc as plsc

_B = 16384
_F = 26
_D = 16
_ND = 13
_H1, _H2 = 128, 64
_NROWS = _B * _F
_WIN = 128
_BN = 1024
_VPAD = 62501


def _sc_gather(emb_table, lin16, idx_flat, lidx_flat):
    mesh = plsc.VectorSubcoreMesh(core_axis_name="c", subcore_axis_name="s")

    @functools.partial(
        pl.kernel,
        out_type=(
            jax.ShapeDtypeStruct((_NROWS, _D), jnp.float32),
            jax.ShapeDtypeStruct((_NROWS, 16), jnp.float32),
        ),
        mesh=mesh,
        compiler_params=pltpu.CompilerParams(use_tc_tiling_on_sc=False),
    )
    def k(emb_hbm, lin_hbm, idx_hbm, lidx_hbm, emb_out_hbm, lin_out_hbm):
        def body(idx_vmem, lidx_vmem, emb_vmem, lin_vmem):
            pltpu.sync_copy(emb_hbm.at[idx_vmem.at[0]], emb_vmem)
            pltpu.sync_copy(lin_hbm.at[lidx_vmem.at[0]], lin_vmem)

        pltpu.emit_pipeline(
            body,
            grid=(_NROWS // _WIN,),
            in_specs=[
                pl.BlockSpec((1, _WIN), lambda i: (0, i)),
                pl.BlockSpec((1, _WIN), lambda i: (0, i)),
            ],
            out_specs=[
                pl.BlockSpec((_WIN, _D), lambda i: (i, 0)),
                pl.BlockSpec((_WIN, 16), lambda i: (i, 0)),
            ],
            core_axis_name=("c", "s"),
            dimension_semantics=(pltpu.PARALLEL,),
        )(idx_hbm, lidx_hbm, emb_out_hbm, lin_out_hbm)

    return k(emb_table, lin16, idx_flat, lidx_flat)


def _tc_body(emb_ref, nb_ref, mod_ref, dense_ref, w1e_ref, w1d_ref, b1_ref,
             w2_ref, b2_ref, wout_ref, dw_ref, c0_ref, out_ref):
    e = emb_ref[...]
    d = dense_ref[...]

    s = e[:, 0:_D]
    for f in range(1, _F):
        s = s + e[:, f * _D:(f + 1) * _D]
    sos = jnp.sum(s * s, axis=1, keepdims=True)
    ssq = jnp.sum(e * e, axis=1, keepdims=True)
    fm = 0.5 * (sos - ssq)

    m = mod_ref[...]
    nb = nb_ref[...]
    lane = lax.broadcasted_iota(jnp.int32, (1, 16), 1)
    lin_sum = jnp.zeros_like(fm)
    for f in range(_F):
        mask = (m[:, f:f + 1] == lane)
        sel = jnp.where(mask, nb[:, f * 16:(f + 1) * 16], 0.0)
        lin_sum = lin_sum + jnp.sum(sel, axis=1, keepdims=True)

    ld = jnp.dot(d, dw_ref[...], preferred_element_type=jnp.float32,
                 precision=lax.Precision.HIGHEST)

    h1 = jnp.dot(e, w1e_ref[...], preferred_element_type=jnp.float32,
                 precision=lax.Precision.HIGHEST)
    h1 = h1 + jnp.dot(d, w1d_ref[...], preferred_element_type=jnp.float32,
                      precision=lax.Precision.HIGHEST)
    h1 = jnp.maximum(h1 + b1_ref[...], 0.0)
    h2 = jnp.dot(h1, w2_ref[...], preferred_element_type=jnp.float32,
                 precision=lax.Precision.HIGHEST)
    h2 = jnp.maximum(h2 + b2_ref[...], 0.0)
    dnn = jnp.sum(h2 * wout_ref[...], axis=1, keepdims=True)

    out_ref[...] = lin_sum + ld + fm + dnn + c0_ref[...]


def _tc_head(embed_flat, lin_nb, mod, dense_x, w1e, w1d, b1, w2, b2,
             wout, dw, c0):
    grid = (_B // _BN,)
    fd = _F * _D
    return pl.pallas_call(
        _tc_body,
        grid=grid,
        in_specs=[
            pl.BlockSpec((_BN, fd), lambda i: (i, 0)),
            pl.BlockSpec((_BN, _F * 16), lambda i: (i, 0)),
            pl.BlockSpec((_BN, _F), lambda i: (i, 0)),
            pl.BlockSpec((_BN, _ND), lambda i: (i, 0)),
            pl.BlockSpec((fd, _H1), lambda i: (0, 0)),
            pl.BlockSpec((_ND, _H1), lambda i: (0, 0)),
            pl.BlockSpec((1, _H1), lambda i: (0, 0)),
            pl.BlockSpec((_H1, _H2), lambda i: (0, 0)),
            pl.BlockSpec((1, _H2), lambda i: (0, 0)),
            pl.BlockSpec((1, _H2), lambda i: (0, 0)),
            pl.BlockSpec((_ND, 1), lambda i: (0, 0)),
            pl.BlockSpec((1, 1), lambda i: (0, 0)),
        ],
        out_specs=pl.BlockSpec((_BN, 1), lambda i: (i, 0)),
        out_shape=jax.ShapeDtypeStruct((_B, 1), jnp.float32),
    )(embed_flat, lin_nb, mod, dense_x, w1e, w1d, b1, w2, b2, wout, dw, c0)


def kernel(sparse_x, dense_x, emb_table, lin_table, bias, dense_w,
           W1, b1, W2, b2, Wout, bout):
    idx = sparse_x.astype(jnp.int32)
    idx_flat = idx.reshape(1, _NROWS)
    lidx_flat = (idx_flat >> 4)
    mod = (idx & 15)

    lin_flat = lin_table.reshape(-1)
    lin16 = jnp.concatenate(
        [lin_flat, jnp.zeros((_VPAD * 16 - lin_flat.shape[0],),
                             jnp.float32)]).reshape(_VPAD, 16)

    emb_rows, lin_nb_rows = _sc_gather(emb_table, lin16, idx_flat, lidx_flat)
    embed_flat = emb_rows.reshape(_B, _F * _D)
    lin_nb = lin_nb_rows.reshape(_B, _F * 16)

    fd = _F * _D
    w1e = W1[:, :fd].T
    w1d = W1[:, fd:].T
    c0 = (bias + bout).reshape(1, 1)
    out = _tc_head(embed_flat, lin_nb, mod, dense_x,
                   w1e, w1d, b1.reshape(1, _H1),
                   W2.T, b2.reshape(1, _H2), Wout.reshape(1, _H2),
                   dense_w.T, c0)
    return out.reshape(_B)


# speedup vs baseline: 1.4033x; 1.4033x over previous
import functools

import jax
import jax.numpy as jnp
from jax import lax
from jax.experimental import pallas as pl
from jax.experimental.pallas import tpu as pltpu
from jax.experimental.pallas import tpu_sc as plsc

_B = 16384
_F = 26
_D = 16
_ND = 13
_H1, _H2 = 128, 64
_NROWS = _B * _F
_WIN = 128
_BN = 1024
_VPAD = 62501
_PH = (8, 8, 8, 2)
_LBLK = _B // _WIN


def _sc_gather(emb_table, lin16, i0, i1, i2, i3, ilin):
    mesh = plsc.VectorSubcoreMesh(core_axis_name="c", subcore_axis_name="s")

    @functools.partial(
        pl.kernel,
        out_type=(
            jax.ShapeDtypeStruct((_B * _PH[0], 16), jnp.float32),
            jax.ShapeDtypeStruct((_B * _PH[1], 16), jnp.float32),
            jax.ShapeDtypeStruct((_B * _PH[2], 16), jnp.float32),
            jax.ShapeDtypeStruct((_B * _PH[3], 16), jnp.float32),
            jax.ShapeDtypeStruct((_LBLK, _WIN), jnp.float32),
        ),
        mesh=mesh,
        scratch_types=[
            pltpu.VMEM_SHARED((_VPAD, 16), jnp.float32),
            pltpu.VMEM((_WIN,), jnp.int32),
            pltpu.VMEM((_WIN, 16), jnp.float32),
        ],
        compiler_params=pltpu.CompilerParams(use_tc_tiling_on_sc=False,
                                             needs_layout_passes=False),
    )
    def k(emb_hbm, lin_hbm, i0_hbm, i1_hbm, i2_hbm, i3_hbm, ilin_hbm,
          e0_hbm, e1_hbm, e2_hbm, e3_hbm, lsum_hbm,
          lin_sp, nb_ref, buf_ref):
        @pl.when(lax.axis_index("s") == 0)
        def _():
            pltpu.sync_copy(lin_hbm, lin_sp)
        plsc.subcore_barrier()

        def emb_body(idx_vmem, out_vmem):
            pltpu.sync_copy(emb_hbm.at[idx_vmem.at[0]], out_vmem)

        def run_phase(idx_hbm, out_hbm, nwin):
            pltpu.emit_pipeline(
                emb_body,
                grid=(nwin,),
                in_specs=[pl.BlockSpec((1, _WIN), lambda i: (0, i))],
                out_specs=[pl.BlockSpec((_WIN, 16), lambda i: (i, 0))],
                core_axis_name=("c", "s"),
                dimension_semantics=(pltpu.PARALLEL,),
            )(idx_hbm, out_hbm)

        run_phase(i0_hbm, e0_hbm, _B * _PH[0] // _WIN)
        run_phase(i1_hbm, e1_hbm, _B * _PH[1] // _WIN)
        run_phase(i2_hbm, e2_hbm, _B * _PH[2] // _WIN)
        run_phase(i3_hbm, e3_hbm, _B * _PH[3] // _WIN)

        def lin_body(ixs, idx_vmem, out_vmem):
            w = ixs[0]
            f = lax.rem(w, _F)
            for c in range(8):
                v = idx_vmem[0, pl.ds(16 * c, 16)]
                nb_ref[pl.ds(16 * c, 16)] = lax.shift_right_logical(v, 4)
            pltpu.sync_copy(lin_sp.at[nb_ref], buf_ref)
            rows = lax.iota(jnp.int32, 16)
            for c in range(8):
                v = idx_vmem[0, pl.ds(16 * c, 16)]
                m = lax.bitwise_and(v, 15)
                sel = plsc.load_gather(buf_ref, [rows + 16 * c, m])
                cur = out_vmem[0, pl.ds(16 * c, 16)]
                out_vmem[0, pl.ds(16 * c, 16)] = jnp.where(f == 0, sel,
                                                           cur + sel)

        pltpu.emit_pipeline(
            lin_body,
            grid=(_LBLK * _F,),
            in_specs=[pl.BlockSpec((1, _WIN), lambda w: (w, 0))],
            out_specs=[pl.BlockSpec((1, _WIN), lambda w: (w // _F, 0))],
            core_axis_name=("c", "s"),
            dimension_semantics=(pltpu.PARALLEL,),
            _explicit_indices=True,
        )(ilin_hbm, lsum_hbm)

    return k(emb_table, lin16, i0, i1, i2, i3, ilin)


def _tc_body(e0_ref, e1_ref, e2_ref, e3_ref, lin_ref, dense_ref,
             w10_ref, w11_ref, w12_ref, w13_ref, w1d_ref, b1_ref,
             w2_ref, b2_ref, wout_ref, dw_ref, c0_ref, out_ref):
    e0, e1, e2, e3 = e0_ref[...], e1_ref[...], e2_ref[...], e3_ref[...]
    d = dense_ref[...]

    parts = [(e0, 8), (e1, 8), (e2, 8), (e3, 2)]
    s = None
    ssq = None
    for e, nf in parts:
        for g in range(nf):
            sl = e[:, 16 * g:16 * (g + 1)]
            s = sl if s is None else s + sl
        q = jnp.sum(e * e, axis=1, keepdims=True)
        ssq = q if ssq is None else ssq + q
    fm = 0.5 * (jnp.sum(s * s, axis=1, keepdims=True) - ssq)

    ld = jnp.dot(d, dw_ref[...], preferred_element_type=jnp.float32,
                 precision=lax.Precision.HIGHEST)

    h1 = jnp.dot(e0, w10_ref[...], preferred_element_type=jnp.float32,
                 precision=lax.Precision.HIGHEST)
    h1 = h1 + jnp.dot(e1, w11_ref[...], preferred_element_type=jnp.float32,
                      precision=lax.Precision.HIGHEST)
    h1 = h1 + jnp.dot(e2, w12_ref[...], preferred_element_type=jnp.float32,
                      precision=lax.Precision.HIGHEST)
    h1 = h1 + jnp.dot(e3, w13_ref[...], preferred_element_type=jnp.float32,
                      precision=lax.Precision.HIGHEST)
    h1 = h1 + jnp.dot(d, w1d_ref[...], preferred_element_type=jnp.float32,
                      precision=lax.Precision.HIGHEST)
    h1 = jnp.maximum(h1 + b1_ref[...], 0.0)
    h2 = jnp.dot(h1, w2_ref[...], preferred_element_type=jnp.float32,
                 precision=lax.Precision.HIGHEST)
    h2 = jnp.maximum(h2 + b2_ref[...], 0.0)
    dnn = jnp.sum(h2 * wout_ref[...], axis=1, keepdims=True)

    out_ref[...] = lin_ref[...] + ld + fm + dnn + c0_ref[...]


def _tc_head(e0, e1, e2, e3, lin, dense_x, w10, w11, w12, w13, w1d, b1,
             w2, b2, wout, dw, c0):
    grid = (_B // _BN,)
    return pl.pallas_call(
        _tc_body,
        grid=grid,
        in_specs=[
            pl.BlockSpec((_BN, 128), lambda i: (i, 0)),
            pl.BlockSpec((_BN, 128), lambda i: (i, 0)),
            pl.BlockSpec((_BN, 128), lambda i: (i, 0)),
            pl.BlockSpec((_BN, 32), lambda i: (i, 0)),
            pl.BlockSpec((_BN, 1), lambda i: (i, 0)),
            pl.BlockSpec((_BN, _ND), lambda i: (i, 0)),
            pl.BlockSpec((128, _H1), lambda i: (0, 0)),
            pl.BlockSpec((128, _H1), lambda i: (0, 0)),
            pl.BlockSpec((128, _H1), lambda i: (0, 0)),
            pl.BlockSpec((32, _H1), lambda i: (0, 0)),
            pl.BlockSpec((_ND, _H1), lambda i: (0, 0)),
            pl.BlockSpec((1, _H1), lambda i: (0, 0)),
            pl.BlockSpec((_H1, _H2), lambda i: (0, 0)),
            pl.BlockSpec((1, _H2), lambda i: (0, 0)),
            pl.BlockSpec((1, _H2), lambda i: (0, 0)),
            pl.BlockSpec((_ND, 1), lambda i: (0, 0)),
            pl.BlockSpec((1, 1), lambda i: (0, 0)),
        ],
        out_specs=pl.BlockSpec((_BN, 1), lambda i: (i, 0)),
        out_shape=jax.ShapeDtypeStruct((_B, 1), jnp.float32),
    )(e0, e1, e2, e3, lin, dense_x, w10, w11, w12, w13, w1d, b1,
      w2, b2, wout, dw, c0)


def kernel(sparse_x, dense_x, emb_table, lin_table, bias, dense_w,
           W1, b1, W2, b2, Wout, bout):
    idx = sparse_x.astype(jnp.int32)

    i0 = idx[:, 0:8].reshape(1, -1)
    i1 = idx[:, 8:16].reshape(1, -1)
    i2 = idx[:, 16:24].reshape(1, -1)
    i3 = idx[:, 24:26].reshape(1, -1)

    ilin = idx.reshape(_LBLK, _WIN, _F).transpose(0, 2, 1).reshape(
        _LBLK * _F, _WIN)

    lin_flat = lin_table.reshape(-1)
    lin16 = jnp.concatenate(
        [lin_flat, jnp.zeros((_VPAD * 16 - lin_flat.shape[0],),
                             jnp.float32)]).reshape(_VPAD, 16)

    e0, e1, e2, e3, lsum = _sc_gather(emb_table, lin16, i0, i1, i2, i3, ilin)
    e0 = e0.reshape(_B, 128)
    e1 = e1.reshape(_B, 128)
    e2 = e2.reshape(_B, 128)
    e3 = e3.reshape(_B, 32)
    lin = lsum.reshape(_B, 1)

    fd = _F * _D
    w1e = W1[:, :fd].T
    w10 = w1e[0:128]
    w11 = w1e[128:256]
    w12 = w1e[256:384]
    w13 = w1e[384:416]
    w1d = W1[:, fd:].T
    c0 = (bias + bout).reshape(1, 1)
    out = _tc_head(e0, e1, e2, e3, lin, dense_x,
                   w10, w11, w12, w13, w1d, b1.reshape(1, _H1),
                   W2.T, b2.reshape(1, _H2), Wout.reshape(1, _H2),
                   dense_w.T, c0)
    return out.reshape(_B)


# speedup vs baseline: 1.6178x; 1.1528x over previous
import functools

import jax
import jax.numpy as jnp
from jax import lax
from jax.experimental import pallas as pl
from jax.experimental.pallas import tpu as pltpu
from jax.experimental.pallas import tpu_sc as plsc

_B = 16384
_F = 26
_D = 16
_ND = 13
_H1, _H2 = 128, 64
_V = 1000012
_NROWS = _B * _F
_WIN = 128
_BN = 1024
_VPAD = 62501
_PH = (8, 8, 8, 2)
_LBLK = _B // _WIN
_TW = 4096
_TSTEPS = -(-_V // _TW)
_VROWS128 = _TSTEPS * _TW // 8


def _transpose_body(in_ref, out_ref, scr_ref):
    scr_ref[...] = in_ref[...].T
    for k in range(8):
        out_ref[:, 16 * k:16 * (k + 1)] = scr_ref[pl.Slice(k, _TW // 8, 8), :]


def _tc_transpose(emb_t):
    return pl.pallas_call(
        _transpose_body,
        grid=(_TSTEPS,),
        in_specs=[pl.BlockSpec((16, _TW), lambda i: (0, i))],
        out_specs=pl.BlockSpec((_TW // 8, 128), lambda i: (i, 0)),
        out_shape=jax.ShapeDtypeStruct((_VROWS128, 128), jnp.float32),
        scratch_shapes=[pltpu.VMEM((_TW, 16), jnp.float32)],
    )(emb_t)


def _sc_linsum(lin16, ilin):
    mesh = plsc.VectorSubcoreMesh(core_axis_name="c", subcore_axis_name="s")

    @functools.partial(
        pl.kernel,
        out_type=jax.ShapeDtypeStruct((_LBLK, _WIN), jnp.float32),
        mesh=mesh,
        scratch_types=[
            pltpu.VMEM_SHARED((_VPAD, 16), jnp.float32),
            pltpu.VMEM((_WIN,), jnp.int32),
            pltpu.VMEM((_WIN, 16), jnp.float32),
        ],
        compiler_params=pltpu.CompilerParams(use_tc_tiling_on_sc=False,
                                             needs_layout_passes=False),
    )
    def k(lin_hbm, ilin_hbm, lsum_hbm, lin_sp, nb_ref, buf_ref):
        @pl.when(lax.axis_index("s") == 0)
        def _():
            pltpu.sync_copy(lin_hbm, lin_sp)
        plsc.subcore_barrier()

        def lin_body(ixs, idx_vmem, out_vmem):
            f = lax.rem(ixs[0], _F)
            for c in range(8):
                v = idx_vmem[0, pl.ds(16 * c, 16)]
                nb_ref[pl.ds(16 * c, 16)] = lax.shift_right_logical(v, 4)
            pltpu.sync_copy(lin_sp.at[nb_ref], buf_ref)
            rows = lax.iota(jnp.int32, 16)
            for c in range(8):
                v = idx_vmem[0, pl.ds(16 * c, 16)]
                m = lax.bitwise_and(v, 15)
                sel = plsc.load_gather(buf_ref, [rows + 16 * c, m])
                cur = out_vmem[0, pl.ds(16 * c, 16)]
                out_vmem[0, pl.ds(16 * c, 16)] = jnp.where(f == 0, sel,
                                                           cur + sel)

        pltpu.emit_pipeline(
            lin_body,
            grid=(_LBLK * _F,),
            in_specs=[pl.BlockSpec((1, _WIN), lambda w: (w, 0))],
            out_specs=[pl.BlockSpec((1, _WIN), lambda w: (w // _F, 0))],
            core_axis_name=("c", "s"),
            dimension_semantics=(pltpu.PARALLEL,),
            _explicit_indices=True,
        )(ilin_hbm, lsum_hbm)

    return k(lin16, ilin)


def _sc_gather(emb_lin, i0, i1, i2, i3):
    mesh = plsc.VectorSubcoreMesh(core_axis_name="c", subcore_axis_name="s")

    @functools.partial(
        pl.kernel,
        out_type=(
            jax.ShapeDtypeStruct((_B * _PH[0], 16), jnp.float32),
            jax.ShapeDtypeStruct((_B * _PH[1], 16), jnp.float32),
            jax.ShapeDtypeStruct((_B * _PH[2], 16), jnp.float32),
            jax.ShapeDtypeStruct((_B * _PH[3], 16), jnp.float32),
        ),
        mesh=mesh,
        compiler_params=pltpu.CompilerParams(use_tc_tiling_on_sc=False,
                                             needs_layout_passes=False),
    )
    def k(emb_hbm, i0_hbm, i1_hbm, i2_hbm, i3_hbm,
          e0_hbm, e1_hbm, e2_hbm, e3_hbm):
        def emb_body(idx_vmem, out_vmem):
            pltpu.sync_copy(emb_hbm.at[idx_vmem.at[0]], out_vmem)

        def run_phase(idx_hbm, out_hbm, nwin):
            pltpu.emit_pipeline(
                emb_body,
                grid=(nwin,),
                in_specs=[pl.BlockSpec((1, _WIN), lambda i: (0, i))],
                out_specs=[pl.BlockSpec((_WIN, 16), lambda i: (i, 0))],
                core_axis_name=("c", "s"),
                dimension_semantics=(pltpu.PARALLEL,),
            )(idx_hbm, out_hbm)

        run_phase(i0_hbm, e0_hbm, _B * _PH[0] // _WIN)
        run_phase(i1_hbm, e1_hbm, _B * _PH[1] // _WIN)
        run_phase(i2_hbm, e2_hbm, _B * _PH[2] // _WIN)
        run_phase(i3_hbm, e3_hbm, _B * _PH[3] // _WIN)

    return k(emb_lin, i0, i1, i2, i3)


def _tc_body(e0_ref, e1_ref, e2_ref, e3_ref, lin_ref, dense_ref,
             w10_ref, w11_ref, w12_ref, w13_ref, w1d_ref, b1_ref,
             w2_ref, b2_ref, wout_ref, dw_ref, c0_ref, out_ref):
    e0, e1, e2, e3 = e0_ref[...], e1_ref[...], e2_ref[...], e3_ref[...]
    d = dense_ref[...]

    parts = [(e0, 8), (e1, 8), (e2, 8), (e3, 2)]
    s = None
    ssq = None
    for e, nf in parts:
        for g in range(nf):
            sl = e[:, 16 * g:16 * (g + 1)]
            s = sl if s is None else s + sl
        q = jnp.sum(e * e, axis=1, keepdims=True)
        ssq = q if ssq is None else ssq + q
    fm = 0.5 * (jnp.sum(s * s, axis=1, keepdims=True) - ssq)

    ld = jnp.dot(d, dw_ref[...], preferred_element_type=jnp.float32,
                 precision=lax.Precision.HIGHEST)

    h1 = jnp.dot(e0, w10_ref[...], preferred_element_type=jnp.float32,
                 precision=lax.Precision.HIGHEST)
    h1 = h1 + jnp.dot(e1, w11_ref[...], preferred_element_type=jnp.float32,
                      precision=lax.Precision.HIGHEST)
    h1 = h1 + jnp.dot(e2, w12_ref[...], preferred_element_type=jnp.float32,
                      precision=lax.Precision.HIGHEST)
    h1 = h1 + jnp.dot(e3, w13_ref[...], preferred_element_type=jnp.float32,
                      precision=lax.Precision.HIGHEST)
    h1 = h1 + jnp.dot(d, w1d_ref[...], preferred_element_type=jnp.float32,
                      precision=lax.Precision.HIGHEST)
    h1 = jnp.maximum(h1 + b1_ref[...], 0.0)
    h2 = jnp.dot(h1, w2_ref[...], preferred_element_type=jnp.float32,
                 precision=lax.Precision.HIGHEST)
    h2 = jnp.maximum(h2 + b2_ref[...], 0.0)
    dnn = jnp.sum(h2 * wout_ref[...], axis=1, keepdims=True)

    out_ref[...] = lin_ref[...] + ld + fm + dnn + c0_ref[...]


def _tc_head(e0, e1, e2, e3, lin, dense_x, w10, w11, w12, w13, w1d, b1,
             w2, b2, wout, dw, c0):
    grid = (_B // _BN,)
    return pl.pallas_call(
        _tc_body,
        grid=grid,
        in_specs=[
            pl.BlockSpec((_BN, 128), lambda i: (i, 0)),
            pl.BlockSpec((_BN, 128), lambda i: (i, 0)),
            pl.BlockSpec((_BN, 128), lambda i: (i, 0)),
            pl.BlockSpec((_BN, 32), lambda i: (i, 0)),
            pl.BlockSpec((_BN, 1), lambda i: (i, 0)),
            pl.BlockSpec((_BN, _ND), lambda i: (i, 0)),
            pl.BlockSpec((128, _H1), lambda i: (0, 0)),
            pl.BlockSpec((128, _H1), lambda i: (0, 0)),
            pl.BlockSpec((128, _H1), lambda i: (0, 0)),
            pl.BlockSpec((32, _H1), lambda i: (0, 0)),
            pl.BlockSpec((_ND, _H1), lambda i: (0, 0)),
            pl.BlockSpec((1, _H1), lambda i: (0, 0)),
            pl.BlockSpec((_H1, _H2), lambda i: (0, 0)),
            pl.BlockSpec((1, _H2), lambda i: (0, 0)),
            pl.BlockSpec((1, _H2), lambda i: (0, 0)),
            pl.BlockSpec((_ND, 1), lambda i: (0, 0)),
            pl.BlockSpec((1, 1), lambda i: (0, 0)),
        ],
        out_specs=pl.BlockSpec((_BN, 1), lambda i: (i, 0)),
        out_shape=jax.ShapeDtypeStruct((_B, 1), jnp.float32),
    )(e0, e1, e2, e3, lin, dense_x, w10, w11, w12, w13, w1d, b1,
      w2, b2, wout, dw, c0)


def kernel(sparse_x, dense_x, emb_table, lin_table, bias, dense_w,
           W1, b1, W2, b2, Wout, bout):
    idx = sparse_x.astype(jnp.int32)

    i0 = idx[:, 0:8].reshape(1, -1)
    i1 = idx[:, 8:16].reshape(1, -1)
    i2 = idx[:, 16:24].reshape(1, -1)
    i3 = idx[:, 24:26].reshape(1, -1)

    ilin = idx.reshape(_LBLK, _WIN, _F).transpose(0, 2, 1).reshape(
        _LBLK * _F, _WIN)

    lin_flat = lin_table.reshape(-1)
    lin16 = jnp.concatenate(
        [lin_flat, jnp.zeros((_VPAD * 16 - lin_flat.shape[0],),
                             jnp.float32)]).reshape(_VPAD, 16)

    emb_lin = _tc_transpose(emb_table.T).reshape(_VROWS128 * 8, 16)

    lsum = _sc_linsum(lin16, ilin)
    e0, e1, e2, e3 = _sc_gather(emb_lin, i0, i1, i2, i3)
    e0 = e0.reshape(_B, 128)
    e1 = e1.reshape(_B, 128)
    e2 = e2.reshape(_B, 128)
    e3 = e3.reshape(_B, 32)
    lin = lsum.reshape(_B, 1)

    fd = _F * _D
    w1e = W1[:, :fd].T
    w10 = w1e[0:128]
    w11 = w1e[128:256]
    w12 = w1e[256:384]
    w13 = w1e[384:416]
    w1d = W1[:, fd:].T
    c0 = (bias + bout).reshape(1, 1)
    out = _tc_head(e0, e1, e2, e3, lin, dense_x,
                   w10, w11, w12, w13, w1d, b1.reshape(1, _H1),
                   W2.T, b2.reshape(1, _H2), Wout.reshape(1, _H2),
                   dense_w.T, c0)
    return out.reshape(_B)


# speedup vs baseline: 1.8055x; 1.1160x over previous
import functools

import jax
import jax.numpy as jnp
from jax import lax
from jax.experimental import pallas as pl
from jax.experimental.pallas import tpu as pltpu
from jax.experimental.pallas import tpu_sc as plsc

_B = 16384
_F = 26
_D = 16
_ND = 13
_H1, _H2 = 128, 64
_V = 1000012
_NROWS = _B * _F
_WIN = 128
_BN = 1024
_VPAD = 62501
_PH = (8, 8, 8, 2)
_LBLK = _B // _WIN
_TW = 8192
_TSTEPS = -(-_V // _TW)
_VROWS128 = _TSTEPS * _TW // 8


def _transpose_body(in_ref, out_ref, scr_ref):
    scr_ref[...] = in_ref[...].T
    for k in range(8):
        out_ref[:, 16 * k:16 * (k + 1)] = scr_ref[pl.Slice(k, _TW // 8, 8), :]


def _tc_transpose(emb_t):
    return pl.pallas_call(
        _transpose_body,
        grid=(_TSTEPS,),
        in_specs=[pl.BlockSpec((16, _TW), lambda i: (0, i))],
        out_specs=pl.BlockSpec((_TW // 8, 128), lambda i: (i, 0)),
        out_shape=jax.ShapeDtypeStruct((_VROWS128, 128), jnp.float32),
        scratch_shapes=[pltpu.VMEM((_TW, 16), jnp.float32)],
    )(emb_t)


def _sc_linsum(lin16, ilin):
    mesh = plsc.VectorSubcoreMesh(core_axis_name="c", subcore_axis_name="s")

    @functools.partial(
        pl.kernel,
        out_type=jax.ShapeDtypeStruct((_LBLK, _WIN), jnp.float32),
        mesh=mesh,
        scratch_types=[
            pltpu.VMEM_SHARED((_VPAD, 16), jnp.float32),
            pltpu.VMEM((_WIN,), jnp.int32),
            pltpu.VMEM((_WIN, 16), jnp.float32),
        ],
        compiler_params=pltpu.CompilerParams(use_tc_tiling_on_sc=False,
                                             needs_layout_passes=False),
    )
    def k(lin_hbm, ilin_hbm, lsum_hbm, lin_sp, nb_ref, buf_ref):
        @pl.when(lax.axis_index("s") == 0)
        def _():
            pltpu.sync_copy(lin_hbm, lin_sp)
        plsc.subcore_barrier()

        def lin_body(ixs, idx_vmem, out_vmem):
            f = lax.rem(ixs[0], _F)
            for c in range(8):
                v = idx_vmem[0, pl.ds(16 * c, 16)]
                nb_ref[pl.ds(16 * c, 16)] = lax.shift_right_logical(v, 4)
            pltpu.sync_copy(lin_sp.at[nb_ref], buf_ref)
            rows = lax.iota(jnp.int32, 16)
            for c in range(8):
                v = idx_vmem[0, pl.ds(16 * c, 16)]
                m = lax.bitwise_and(v, 15)
                sel = plsc.load_gather(buf_ref, [rows + 16 * c, m])
                cur = out_vmem[0, pl.ds(16 * c, 16)]
                out_vmem[0, pl.ds(16 * c, 16)] = jnp.where(f == 0, sel,
                                                           cur + sel)

        pltpu.emit_pipeline(
            lin_body,
            grid=(_LBLK * _F,),
            in_specs=[pl.BlockSpec((1, _WIN), lambda w: (w, 0))],
            out_specs=[pl.BlockSpec((1, _WIN), lambda w: (w // _F, 0))],
            core_axis_name=("c", "s"),
            dimension_semantics=(pltpu.PARALLEL,),
            _explicit_indices=True,
        )(ilin_hbm, lsum_hbm)

    return k(lin16, ilin)


def _sc_gather(emb_lin, i0, i1, i2, i3, nb):
    mesh = plsc.VectorSubcoreMesh(core_axis_name="c", subcore_axis_name="s")

    @functools.partial(
        pl.kernel,
        out_type=(
            jax.ShapeDtypeStruct((nb * _PH[0], 16), jnp.float32),
            jax.ShapeDtypeStruct((nb * _PH[1], 16), jnp.float32),
            jax.ShapeDtypeStruct((nb * _PH[2], 16), jnp.float32),
            jax.ShapeDtypeStruct((nb * _PH[3], 16), jnp.float32),
        ),
        mesh=mesh,
        compiler_params=pltpu.CompilerParams(use_tc_tiling_on_sc=False,
                                             needs_layout_passes=False),
    )
    def k(emb_hbm, i0_hbm, i1_hbm, i2_hbm, i3_hbm,
          e0_hbm, e1_hbm, e2_hbm, e3_hbm):
        def emb_body(idx_vmem, out_vmem):
            pltpu.sync_copy(emb_hbm.at[idx_vmem.at[0]], out_vmem)

        def run_phase(idx_hbm, out_hbm, nwin):
            pltpu.emit_pipeline(
                emb_body,
                grid=(nwin,),
                in_specs=[pl.BlockSpec((1, _WIN), lambda i: (0, i))],
                out_specs=[pl.BlockSpec((_WIN, 16), lambda i: (i, 0))],
                core_axis_name=("c", "s"),
                dimension_semantics=(pltpu.PARALLEL,),
            )(idx_hbm, out_hbm)

        run_phase(i0_hbm, e0_hbm, nb * _PH[0] // _WIN)
        run_phase(i1_hbm, e1_hbm, nb * _PH[1] // _WIN)
        run_phase(i2_hbm, e2_hbm, nb * _PH[2] // _WIN)
        run_phase(i3_hbm, e3_hbm, nb * _PH[3] // _WIN)

    return k(emb_lin, i0, i1, i2, i3)


_DOT_PREC = lax.Precision.DEFAULT


def _dot(a, b):
    return lax.dot_general(a, b, (((1,), (0,)), ((), ())),
                           precision=_DOT_PREC,
                           preferred_element_type=jnp.float32)


def _tc_body(e0_ref, e1_ref, e2_ref, e3_ref, lin_ref, dense_ref,
             w10_ref, w11_ref, w12_ref, w13_ref, w1d_ref, b1_ref,
             w2_ref, b2_ref, wout_ref, dw_ref, c0_ref, out_ref):
    e0, e1, e2, e3 = e0_ref[...], e1_ref[...], e2_ref[...], e3_ref[...]
    d = dense_ref[...]

    parts = [(e0, 8), (e1, 8), (e2, 8), (e3, 2)]
    s = None
    ssq = None
    for e, nf in parts:
        for g in range(nf):
            sl = e[:, 16 * g:16 * (g + 1)]
            s = sl if s is None else s + sl
        q = jnp.sum(e * e, axis=1, keepdims=True)
        ssq = q if ssq is None else ssq + q
    fm = 0.5 * (jnp.sum(s * s, axis=1, keepdims=True) - ssq)

    ld = _dot(d, dw_ref[...])

    h1 = _dot(e0, w10_ref[...])
    h1 = h1 + _dot(e1, w11_ref[...])
    h1 = h1 + _dot(e2, w12_ref[...])
    h1 = h1 + _dot(e3, w13_ref[...])
    h1 = h1 + _dot(d, w1d_ref[...])
    h1 = jnp.maximum(h1 + b1_ref[...], 0.0)
    h2 = _dot(h1, w2_ref[...])
    h2 = jnp.maximum(h2 + b2_ref[...], 0.0)
    dnn = jnp.sum(h2 * wout_ref[...], axis=1, keepdims=True)

    out_ref[...] = lin_ref[...] + ld + fm + dnn + c0_ref[...]


def _tc_head(e0, e1, e2, e3, lin, dense_x, w10, w11, w12, w13, w1d, b1,
             w2, b2, wout, dw, c0, nb):
    grid = (nb // _BN,)
    return pl.pallas_call(
        _tc_body,
        grid=grid,
        in_specs=[
            pl.BlockSpec((_BN, 128), lambda i: (i, 0)),
            pl.BlockSpec((_BN, 128), lambda i: (i, 0)),
            pl.BlockSpec((_BN, 128), lambda i: (i, 0)),
            pl.BlockSpec((_BN, 32), lambda i: (i, 0)),
            pl.BlockSpec((_BN, 1), lambda i: (i, 0)),
            pl.BlockSpec((_BN, _ND), lambda i: (i, 0)),
            pl.BlockSpec((128, _H1), lambda i: (0, 0)),
            pl.BlockSpec((128, _H1), lambda i: (0, 0)),
            pl.BlockSpec((128, _H1), lambda i: (0, 0)),
            pl.BlockSpec((32, _H1), lambda i: (0, 0)),
            pl.BlockSpec((_ND, _H1), lambda i: (0, 0)),
            pl.BlockSpec((1, _H1), lambda i: (0, 0)),
            pl.BlockSpec((_H1, _H2), lambda i: (0, 0)),
            pl.BlockSpec((1, _H2), lambda i: (0, 0)),
            pl.BlockSpec((1, _H2), lambda i: (0, 0)),
            pl.BlockSpec((_ND, 1), lambda i: (0, 0)),
            pl.BlockSpec((1, 1), lambda i: (0, 0)),
        ],
        out_specs=pl.BlockSpec((_BN, 1), lambda i: (i, 0)),
        out_shape=jax.ShapeDtypeStruct((nb, 1), jnp.float32),
    )(e0, e1, e2, e3, lin, dense_x, w10, w11, w12, w13, w1d, b1,
      w2, b2, wout, dw, c0)


def kernel(sparse_x, dense_x, emb_table, lin_table, bias, dense_w,
           W1, b1, W2, b2, Wout, bout):
    idx = sparse_x.astype(jnp.int32)

    ilin = idx.reshape(_LBLK, _WIN, _F).transpose(0, 2, 1).reshape(
        _LBLK * _F, _WIN)

    lin_flat = lin_table.reshape(-1)
    lin16 = jnp.concatenate(
        [lin_flat, jnp.zeros((_VPAD * 16 - lin_flat.shape[0],),
                             jnp.float32)]).reshape(_VPAD, 16)

    emb_lin = _tc_transpose(emb_table.T).reshape(_VROWS128 * 8, 16)

    lsum = _sc_linsum(lin16, ilin)
    lin = lsum.reshape(_B, 1)

    fd = _F * _D
    w1e = W1[:, :fd].T
    w10 = w1e[0:128]
    w11 = w1e[128:256]
    w12 = w1e[256:384]
    w13 = w1e[384:416]
    w1d = W1[:, fd:].T
    c0 = (bias + bout).reshape(1, 1)

    nch, cb = 4, _B // 4
    outs = []
    for c in range(nch):
        sl = slice(c * cb, (c + 1) * cb)
        ic = idx[sl]
        e0, e1, e2, e3 = _sc_gather(
            emb_lin,
            ic[:, 0:8].reshape(1, -1), ic[:, 8:16].reshape(1, -1),
            ic[:, 16:24].reshape(1, -1), ic[:, 24:26].reshape(1, -1), cb)
        outs.append(_tc_head(
            e0.reshape(cb, 128), e1.reshape(cb, 128), e2.reshape(cb, 128),
            e3.reshape(cb, 32), lin[sl], dense_x[sl],
            w10, w11, w12, w13, w1d, b1.reshape(1, _H1),
            W2.T, b2.reshape(1, _H2), Wout.reshape(1, _H2),
            dense_w.T, c0, cb))
    return jnp.concatenate(outs, axis=0).reshape(_B)


# speedup vs baseline: 1.8220x; 1.0092x over previous
import functools

import jax
import jax.numpy as jnp
from jax import lax
from jax.experimental import pallas as pl
from jax.experimental.pallas import tpu as pltpu
from jax.experimental.pallas import tpu_sc as plsc

_B = 16384
_F = 26
_D = 16
_ND = 13
_H1, _H2 = 128, 64
_V = 1000012
_NROWS = _B * _F
_WIN = 128
_BN = 1024
_VPAD = 62976
_PH = (8, 8, 8, 2)
_LBLK = _B // _WIN
_TW = 8192
_TSTEPS = -(-_V // _TW)
_VROWS128 = _TSTEPS * _TW // 8


def _transpose_body(in_ref, lin_ref, out_ref, lout_ref, scr_ref):
    scr_ref[...] = in_ref[...].T
    for k in range(8):
        out_ref[:, 16 * k:16 * (k + 1)] = scr_ref[pl.Slice(k, _TW // 8, 8), :]
    lout_ref[0] = lin_ref[...]


def _tc_transpose(emb_t, lin_t):
    return pl.pallas_call(
        _transpose_body,
        grid=(_TSTEPS,),
        in_specs=[pl.BlockSpec((16, _TW), lambda i: (0, i)),
                  pl.BlockSpec((1, _TW), lambda i: (0, i))],
        out_specs=[pl.BlockSpec((_TW // 8, 128), lambda i: (i, 0)),
                   pl.BlockSpec((1, 1, _TW), lambda i: (i, 0, 0))],
        out_shape=[jax.ShapeDtypeStruct((_VROWS128, 128), jnp.float32),
                   jax.ShapeDtypeStruct((_TSTEPS, 1, _TW), jnp.float32)],
        scratch_shapes=[pltpu.VMEM((_TW, 16), jnp.float32)],
    )(emb_t, lin_t)


def _sc_linsum(lin16, ilin):
    mesh = plsc.VectorSubcoreMesh(core_axis_name="c", subcore_axis_name="s")

    @functools.partial(
        pl.kernel,
        out_type=jax.ShapeDtypeStruct((_LBLK, _WIN), jnp.float32),
        mesh=mesh,
        scratch_types=[
            pltpu.VMEM_SHARED((_VPAD, 16), jnp.float32),
            pltpu.VMEM((_WIN,), jnp.int32),
            pltpu.VMEM((_WIN, 16), jnp.float32),
        ],
        compiler_params=pltpu.CompilerParams(use_tc_tiling_on_sc=False,
                                             needs_layout_passes=False),
    )
    def k(lin_hbm, ilin_hbm, lsum_hbm, lin_sp, nb_ref, buf_ref):
        @pl.when(lax.axis_index("s") == 0)
        def _():
            pltpu.sync_copy(lin_hbm, lin_sp)
        plsc.subcore_barrier()

        def lin_body(ixs, idx_vmem, out_vmem):
            f = lax.rem(ixs[0], _F)
            for c in range(8):
                v = idx_vmem[0, pl.ds(16 * c, 16)]
                nb_ref[pl.ds(16 * c, 16)] = lax.shift_right_logical(v, 4)
            pltpu.sync_copy(lin_sp.at[nb_ref], buf_ref)
            rows = lax.iota(jnp.int32, 16)
            for c in range(8):
                v = idx_vmem[0, pl.ds(16 * c, 16)]
                m = lax.bitwise_and(v, 15)
                sel = plsc.load_gather(buf_ref, [rows + 16 * c, m])
                cur = out_vmem[0, pl.ds(16 * c, 16)]
                out_vmem[0, pl.ds(16 * c, 16)] = jnp.where(f == 0, sel,
                                                           cur + sel)

        pltpu.emit_pipeline(
            lin_body,
            grid=(_LBLK * _F,),
            in_specs=[pl.BlockSpec((1, _WIN), lambda w: (w, 0))],
            out_specs=[pl.BlockSpec((1, _WIN), lambda w: (w // _F, 0))],
            core_axis_name=("c", "s"),
            dimension_semantics=(pltpu.PARALLEL,),
            _explicit_indices=True,
        )(ilin_hbm, lsum_hbm)

    return k(lin16, ilin)


def _sc_gather(emb_lin, i0, i1, i2, i3, nb):
    mesh = plsc.VectorSubcoreMesh(core_axis_name="c", subcore_axis_name="s")

    @functools.partial(
        pl.kernel,
        out_type=(
            jax.ShapeDtypeStruct((nb * _PH[0], 16), jnp.float32),
            jax.ShapeDtypeStruct((nb * _PH[1], 16), jnp.float32),
            jax.ShapeDtypeStruct((nb * _PH[2], 16), jnp.float32),
            jax.ShapeDtypeStruct((nb * _PH[3], 16), jnp.float32),
        ),
        mesh=mesh,
        compiler_params=pltpu.CompilerParams(use_tc_tiling_on_sc=False,
                                             needs_layout_passes=False),
    )
    def k(emb_hbm, i0_hbm, i1_hbm, i2_hbm, i3_hbm,
          e0_hbm, e1_hbm, e2_hbm, e3_hbm):
        def emb_body(idx_vmem, out_vmem):
            pltpu.sync_copy(emb_hbm.at[idx_vmem.at[0]], out_vmem)

        def run_phase(idx_hbm, out_hbm, nwin):
            pltpu.emit_pipeline(
                emb_body,
                grid=(nwin,),
                in_specs=[pl.BlockSpec((1, _WIN), lambda i: (0, i))],
                out_specs=[pl.BlockSpec((_WIN, 16), lambda i: (i, 0))],
                core_axis_name=("c", "s"),
                dimension_semantics=(pltpu.PARALLEL,),
            )(idx_hbm, out_hbm)

        run_phase(i0_hbm, e0_hbm, nb * _PH[0] // _WIN)
        run_phase(i1_hbm, e1_hbm, nb * _PH[1] // _WIN)
        run_phase(i2_hbm, e2_hbm, nb * _PH[2] // _WIN)
        run_phase(i3_hbm, e3_hbm, nb * _PH[3] // _WIN)

    return k(emb_lin, i0, i1, i2, i3)


_DOT_PREC = lax.Precision.DEFAULT


def _dot(a, b):
    return lax.dot_general(a, b, (((1,), (0,)), ((), ())),
                           precision=_DOT_PREC,
                           preferred_element_type=jnp.float32)


def _tc_body(e0_ref, e1_ref, e2_ref, e3_ref, lin_ref, dense_ref,
             w10_ref, w11_ref, w12_ref, w13_ref, w1d_ref, b1_ref,
             w2_ref, b2_ref, wout_ref, dw_ref, c0_ref, out_ref):
    e0, e1, e2, e3 = e0_ref[...], e1_ref[...], e2_ref[...], e3_ref[...]
    d = dense_ref[...]

    parts = [(e0, 8), (e1, 8), (e2, 8), (e3, 2)]
    s = None
    ssq = None
    for e, nf in parts:
        for g in range(nf):
            sl = e[:, 16 * g:16 * (g + 1)]
            s = sl if s is None else s + sl
        q = jnp.sum(e * e, axis=1, keepdims=True)
        ssq = q if ssq is None else ssq + q
    fm = 0.5 * (jnp.sum(s * s, axis=1, keepdims=True) - ssq)

    ld = _dot(d, dw_ref[...])

    h1 = _dot(e0, w10_ref[...])
    h1 = h1 + _dot(e1, w11_ref[...])
    h1 = h1 + _dot(e2, w12_ref[...])
    h1 = h1 + _dot(e3, w13_ref[...])
    h1 = h1 + _dot(d, w1d_ref[...])
    h1 = jnp.maximum(h1 + b1_ref[...], 0.0)
    h2 = _dot(h1, w2_ref[...])
    h2 = jnp.maximum(h2 + b2_ref[...], 0.0)
    dnn = jnp.sum(h2 * wout_ref[...], axis=1, keepdims=True)

    out_ref[...] = lin_ref[...] + ld + fm + dnn + c0_ref[...]


def _tc_head(e0, e1, e2, e3, lin, dense_x, w10, w11, w12, w13, w1d, b1,
             w2, b2, wout, dw, c0, nb):
    grid = (nb // _BN,)
    return pl.pallas_call(
        _tc_body,
        grid=grid,
        in_specs=[
            pl.BlockSpec((_BN, 128), lambda i: (i, 0)),
            pl.BlockSpec((_BN, 128), lambda i: (i, 0)),
            pl.BlockSpec((_BN, 128), lambda i: (i, 0)),
            pl.BlockSpec((_BN, 32), lambda i: (i, 0)),
            pl.BlockSpec((_BN, 1), lambda i: (i, 0)),
            pl.BlockSpec((_BN, _ND), lambda i: (i, 0)),
            pl.BlockSpec((128, _H1), lambda i: (0, 0)),
            pl.BlockSpec((128, _H1), lambda i: (0, 0)),
            pl.BlockSpec((128, _H1), lambda i: (0, 0)),
            pl.BlockSpec((32, _H1), lambda i: (0, 0)),
            pl.BlockSpec((_ND, _H1), lambda i: (0, 0)),
            pl.BlockSpec((1, _H1), lambda i: (0, 0)),
            pl.BlockSpec((_H1, _H2), lambda i: (0, 0)),
            pl.BlockSpec((1, _H2), lambda i: (0, 0)),
            pl.BlockSpec((1, _H2), lambda i: (0, 0)),
            pl.BlockSpec((_ND, 1), lambda i: (0, 0)),
            pl.BlockSpec((1, 1), lambda i: (0, 0)),
        ],
        out_specs=pl.BlockSpec((_BN, 1), lambda i: (i, 0)),
        out_shape=jax.ShapeDtypeStruct((nb, 1), jnp.float32),
    )(e0, e1, e2, e3, lin, dense_x, w10, w11, w12, w13, w1d, b1,
      w2, b2, wout, dw, c0)


def kernel(sparse_x, dense_x, emb_table, lin_table, bias, dense_w,
           W1, b1, W2, b2, Wout, bout):
    idx = sparse_x.astype(jnp.int32)

    ilin = idx.reshape(_LBLK, _WIN, _F).transpose(0, 2, 1).reshape(
        _LBLK * _F, _WIN)

    emb128, lin_pad = _tc_transpose(emb_table.T, lin_table.T)
    emb_lin = emb128.reshape(_VROWS128 * 8, 16)
    lin16 = lin_pad.reshape(_VPAD, 16)

    lsum = _sc_linsum(lin16, ilin)
    lin = lsum.reshape(_B, 1)

    fd = _F * _D
    w1e = W1[:, :fd].T
    w10 = w1e[0:128]
    w11 = w1e[128:256]
    w12 = w1e[256:384]
    w13 = w1e[384:416]
    w1d = W1[:, fd:].T
    c0 = (bias + bout).reshape(1, 1)

    nch, cb = 2, _B // 2
    outs = []
    for c in range(nch):
        sl = slice(c * cb, (c + 1) * cb)
        ic = idx[sl]
        e0, e1, e2, e3 = _sc_gather(
            emb_lin,
            ic[:, 0:8].reshape(1, -1), ic[:, 8:16].reshape(1, -1),
            ic[:, 16:24].reshape(1, -1), ic[:, 24:26].reshape(1, -1), cb)
        outs.append(_tc_head(
            e0.reshape(cb, 128), e1.reshape(cb, 128), e2.reshape(cb, 128),
            e3.reshape(cb, 32), lin[sl], dense_x[sl],
            w10, w11, w12, w13, w1d, b1.reshape(1, _H1),
            W2.T, b2.reshape(1, _H2), Wout.reshape(1, _H2),
            dense_w.T, c0, cb))
    return jnp.concatenate(outs, axis=0).reshape(_B)
